# baseline (speedup 1.0000x reference)
import functools

import jax
import jax.numpy as jnp
from jax import lax
from jax.experimental import pallas as pl
from jax.experimental.pallas import tpu as pltpu
from jax.experimental.pallas import tpu_sc as plsc

N = 10000
F = 128
T = 4
E = 320000

NC = 2
NS = 16
NW = NC * NS
CH = 96
CPW0 = 70
CPW1 = 140
CPW_MAX = max(CPW0, CPW1)
E0 = NS * CH * CPW0
E_PAD = NS * CH * (CPW0 + CPW1)
A_ROWS = 10112
RPT = A_ROWS // NS
CC = 96
GA = 10


def _prep_body(h_ref, w_ref, b_ref, src_ref, et_ref, dst_ref, tab_ref, pk_ref):
    hb = h_ref[...]
    for t in range(T):
        tab_ref[t] = lax.dot_general(
            hb, w_ref[t], (((1,), (1,)), ((), ())),
            preferred_element_type=jnp.float32) + b_ref[t]
    pk_ref[...] = (et_ref[...] * N + src_ref[...]) | (dst_ref[...] << 16)


_prep_call = pl.pallas_call(
    _prep_body,
    grid=(GA,),
    in_specs=[
        pl.BlockSpec((N // GA, F), lambda i: (i, 0)),
        pl.BlockSpec((T, F, F), lambda i: (0, 0, 0)),
        pl.BlockSpec((T, F), lambda i: (0, 0)),
        pl.BlockSpec((1, 1, E // GA), lambda i: (i, 0, 0)),
        pl.BlockSpec((1, 1, E // GA), lambda i: (i, 0, 0)),
        pl.BlockSpec((1, 1, E // GA), lambda i: (i, 0, 0)),
    ],
    out_specs=[
        pl.BlockSpec((T, N // GA, F), lambda i: (0, i, 0)),
        pl.BlockSpec((1, 1, E // GA), lambda i: (i, 0, 0)),
    ],
    out_shape=[
        jax.ShapeDtypeStruct((T, N, F), jnp.float32),
        jax.ShapeDtypeStruct((GA, 1, E // GA), jnp.int32),
    ],
)


@functools.partial(
    pl.kernel,
    out_type=jax.ShapeDtypeStruct((NC, A_ROWS, F), jnp.float32),
    mesh=plsc.VectorSubcoreMesh(core_axis_name="c", subcore_axis_name="s"),
    scratch_types=[
        pltpu.VMEM((CPW_MAX, CH), jnp.int32),
        pltpu.VMEM((2, CH), jnp.int32),
        pltpu.VMEM((2, CH), jnp.int32),
        pltpu.VMEM((CH, F), jnp.float32),
        pltpu.VMEM((CH, F), jnp.float32),
        pltpu.VMEM_SHARED((A_ROWS, F), jnp.float32),
        pltpu.SemaphoreType.DMA,
        pltpu.SemaphoreType.DMA,
        pltpu.SemaphoreType.DMA,
    ],
)
def _edge_kernel(tab_hbm, pk_hbm, out_hbm,
                 pk_v, idxb, dstb, rows0, rows1, acc_s, gsem0, gsem1, ssem):
    cid = lax.axis_index("c")
    sid = lax.axis_index("s")
    base = sid * RPT
    nfull = RPT // CC
    rem = RPT - nfull * CC

    pltpu.async_copy(pk_hbm.at[cid, sid], pk_v, gsem0)

    def _zrow(i, carry):
        for j in range(F // 16):
            rows0[i, pl.ds(j * 16, 16)] = jnp.zeros((16,), jnp.float32)
        return carry
    lax.fori_loop(0, CC, _zrow, 0)
    for m in range(nfull):
        pltpu.sync_copy(rows0, acc_s.at[pl.ds(base + m * CC, CC)])
    pltpu.sync_copy(rows0.at[pl.ds(0, rem)],
                    acc_s.at[pl.ds(base + nfull * CC, rem)])

    pltpu.make_async_copy(pk_hbm.at[cid, sid], pk_v, gsem0).wait()
    plsc.subcore_barrier()

    def _pair(k, carry):
        c0 = 2 * k
        for q in range(2):
            for j in range(CH // 16):
                sl = pl.ds(j * 16, 16)
                pk = pk_v[c0 + q, sl]
                idxb[q, sl] = pk & 0xFFFF
                dstb[q, sl] = lax.shift_right_logical(pk, 16)
        pltpu.async_copy(tab_hbm.at[idxb.at[0]], rows0, gsem0)
        pltpu.async_copy(tab_hbm.at[idxb.at[1]], rows1, gsem1)
        pltpu.make_async_copy(tab_hbm.at[idxb.at[0]], rows0, gsem0).wait()
        pltpu.async_copy(rows0, acc_s.at[dstb.at[0]], ssem, add=True)
        pltpu.make_async_copy(tab_hbm.at[idxb.at[1]], rows1, gsem1).wait()
        pltpu.sync_copy(rows1, acc_s.at[dstb.at[1]], add=True)
        pltpu.make_async_copy(rows0, acc_s.at[dstb.at[0]], ssem).wait()
        return carry
    npairs = lax.select(cid == 0, CPW0 // 2, CPW1 // 2)
    lax.fori_loop(0, npairs, _pair, 0)
    plsc.subcore_barrier()

    for m in range(nfull):
        r0 = base + m * CC
        pltpu.sync_copy(acc_s.at[pl.ds(r0, CC)], rows0)
        pltpu.sync_copy(rows0, out_hbm.at[cid, pl.ds(r0, CC)])
    pltpu.sync_copy(acc_s.at[pl.ds(base + nfull * CC, rem)],
                    rows0.at[pl.ds(0, rem)])
    pltpu.sync_copy(rows0.at[pl.ds(0, rem)],
                    out_hbm.at[cid, pl.ds(base + nfull * CC, rem)])


def _combine_body(p_ref, o_ref):
    o_ref[...] = p_ref[0] + p_ref[1]


_combine_call = pl.pallas_call(
    _combine_body,
    grid=(GA,),
    in_specs=[pl.BlockSpec((NC, N // GA, F), lambda i: (0, i, 0))],
    out_specs=pl.BlockSpec((N // GA, F), lambda i: (i, 0)),
    out_shape=jax.ShapeDtypeStruct((N, F), jnp.float32),
)


def kernel(feat, edge_index, etypes, W, b):
    src = edge_index[0]
    dst = edge_index[1]
    tab4, pk3 = _prep_call(
        feat, W, b,
        src.reshape(GA, 1, E // GA), etypes.reshape(GA, 1, E // GA),
        dst.reshape(GA, 1, E // GA))
    table = tab4.reshape(T * N, F)
    pk_flat = pk3.reshape(-1)
    dummy = jnp.int32(N << 16)
    pk0 = pk_flat[:E0].reshape(NS, CPW0, CH)
    pk0 = jnp.concatenate(
        [pk0, jnp.full((NS, CPW_MAX - CPW0, CH), dummy, jnp.int32)], axis=1)
    pk1 = jnp.concatenate(
        [pk_flat[E0:], jnp.full((E_PAD - E,), dummy, jnp.int32)]
    ).reshape(NS, CPW1, CH)
    pk_p = jnp.stack([pk0, pk1])
    partial = _edge_kernel(table, pk_p)
    return _combine_call(partial)


# speedup vs baseline: 1.3044x; 1.3044x over previous
import functools

import jax
import jax.numpy as jnp
from jax import lax
from jax.experimental import pallas as pl
from jax.experimental.pallas import tpu as pltpu
from jax.experimental.pallas import tpu_sc as plsc

N = 10000
F = 128
T = 4
E = 320000

NC = 2
NS = 16
NW = NC * NS
CH = 96
CPW0 = 140
CPW1 = 70
CPW_MAX = max(CPW0, CPW1)
E0 = NS * CH * CPW0
E_PAD = NS * CH * (CPW0 + CPW1)
A_ROWS = 10112
RPT = A_ROWS // NS
CC = 96
GA = 10


def _prep_body(h_ref, w_ref, b_ref, src_ref, et_ref, dst_ref, tab_ref, pk_ref):
    hb = h_ref[...]
    for t in range(T):
        tab_ref[t] = lax.dot_general(
            hb, w_ref[t], (((1,), (1,)), ((), ())),
            preferred_element_type=jnp.float32) + b_ref[t]
    pk_ref[...] = (et_ref[...] * N + src_ref[...]) | (dst_ref[...] << 16)


_prep_call = pl.pallas_call(
    _prep_body,
    grid=(GA,),
    in_specs=[
        pl.BlockSpec((N // GA, F), lambda i: (i, 0)),
        pl.BlockSpec((T, F, F), lambda i: (0, 0, 0)),
        pl.BlockSpec((T, F), lambda i: (0, 0)),
        pl.BlockSpec((1, 1, E // GA), lambda i: (i, 0, 0)),
        pl.BlockSpec((1, 1, E // GA), lambda i: (i, 0, 0)),
        pl.BlockSpec((1, 1, E // GA), lambda i: (i, 0, 0)),
    ],
    out_specs=[
        pl.BlockSpec((T, N // GA, F), lambda i: (0, i, 0)),
        pl.BlockSpec((1, 1, E // GA), lambda i: (i, 0, 0)),
    ],
    out_shape=[
        jax.ShapeDtypeStruct((T, N, F), jnp.float32),
        jax.ShapeDtypeStruct((GA, 1, E // GA), jnp.int32),
    ],
)


@functools.partial(
    pl.kernel,
    out_type=jax.ShapeDtypeStruct((NC, A_ROWS, F), jnp.float32),
    mesh=plsc.VectorSubcoreMesh(core_axis_name="c", subcore_axis_name="s"),
    scratch_types=[
        pltpu.VMEM((CPW_MAX, CH), jnp.int32),
        pltpu.VMEM((2, CH), jnp.int32),
        pltpu.VMEM((2, CH), jnp.int32),
        pltpu.VMEM((CH, F), jnp.float32),
        pltpu.VMEM((CH, F), jnp.float32),
        pltpu.VMEM_SHARED((A_ROWS, F), jnp.float32),
        pltpu.SemaphoreType.DMA,
        pltpu.SemaphoreType.DMA,
        pltpu.SemaphoreType.DMA,
    ],
)
def _edge_kernel(tab_hbm, pk_hbm, out_hbm,
                 pk_v, idxb, dstb, rows0, rows1, acc_s, gsem0, gsem1, ssem):
    cid = lax.axis_index("c")
    sid = lax.axis_index("s")
    base = sid * RPT
    nfull = RPT // CC
    rem = RPT - nfull * CC

    pltpu.async_copy(pk_hbm.at[cid, sid], pk_v, gsem0)

    def _zrow(i, carry):
        for j in range(F // 16):
            rows0[i, pl.ds(j * 16, 16)] = jnp.zeros((16,), jnp.float32)
        return carry
    lax.fori_loop(0, CC, _zrow, 0)
    for m in range(nfull):
        pltpu.sync_copy(rows0, acc_s.at[pl.ds(base + m * CC, CC)])
    pltpu.sync_copy(rows0.at[pl.ds(0, rem)],
                    acc_s.at[pl.ds(base + nfull * CC, rem)])

    pltpu.make_async_copy(pk_hbm.at[cid, sid], pk_v, gsem0).wait()
    plsc.subcore_barrier()

    def _pair(k, carry):
        c0 = 2 * k
        for q in range(2):
            for j in range(CH // 16):
                sl = pl.ds(j * 16, 16)
                pk = pk_v[c0 + q, sl]
                idxb[q, sl] = pk & 0xFFFF
                dstb[q, sl] = lax.shift_right_logical(pk, 16)
        pltpu.async_copy(tab_hbm.at[idxb.at[0]], rows0, gsem0)
        pltpu.async_copy(tab_hbm.at[idxb.at[1]], rows1, gsem1)
        pltpu.make_async_copy(tab_hbm.at[idxb.at[0]], rows0, gsem0).wait()
        pltpu.async_copy(rows0, acc_s.at[dstb.at[0]], ssem, add=True)
        pltpu.make_async_copy(tab_hbm.at[idxb.at[1]], rows1, gsem1).wait()
        pltpu.sync_copy(rows1, acc_s.at[dstb.at[1]], add=True)
        pltpu.make_async_copy(rows0, acc_s.at[dstb.at[0]], ssem).wait()
        return carry
    npairs = lax.select(cid == 0, CPW0 // 2, CPW1 // 2)
    lax.fori_loop(0, npairs, _pair, 0)
    plsc.subcore_barrier()

    for m in range(nfull):
        r0 = base + m * CC
        pltpu.sync_copy(acc_s.at[pl.ds(r0, CC)], rows0)
        pltpu.sync_copy(rows0, out_hbm.at[cid, pl.ds(r0, CC)])
    pltpu.sync_copy(acc_s.at[pl.ds(base + nfull * CC, rem)],
                    rows0.at[pl.ds(0, rem)])
    pltpu.sync_copy(rows0.at[pl.ds(0, rem)],
                    out_hbm.at[cid, pl.ds(base + nfull * CC, rem)])


def _combine_body(p_ref, o_ref):
    o_ref[...] = p_ref[0] + p_ref[1]


_combine_call = pl.pallas_call(
    _combine_body,
    grid=(GA,),
    in_specs=[pl.BlockSpec((NC, N // GA, F), lambda i: (0, i, 0))],
    out_specs=pl.BlockSpec((N // GA, F), lambda i: (i, 0)),
    out_shape=jax.ShapeDtypeStruct((N, F), jnp.float32),
)


def kernel(feat, edge_index, etypes, W, b):
    src = edge_index[0]
    dst = edge_index[1]
    tab4, pk3 = _prep_call(
        feat, W, b,
        src.reshape(GA, 1, E // GA), etypes.reshape(GA, 1, E // GA),
        dst.reshape(GA, 1, E // GA))
    table = tab4.reshape(T * N, F)
    pk_flat = pk3.reshape(-1)
    dummy = jnp.int32(N << 16)
    pk0 = pk_flat[:E0].reshape(NS, CPW0, CH)
    if CPW_MAX > CPW0:
        pk0 = jnp.concatenate(
            [pk0, jnp.full((NS, CPW_MAX - CPW0, CH), dummy, jnp.int32)], axis=1)
    pk1 = jnp.concatenate(
        [pk_flat[E0:], jnp.full((E_PAD - E,), dummy, jnp.int32)]
    ).reshape(NS, CPW1, CH)
    if CPW_MAX > CPW1:
        pk1 = jnp.concatenate(
            [pk1, jnp.full((NS, CPW_MAX - CPW1, CH), dummy, jnp.int32)], axis=1)
    pk_p = jnp.stack([pk0, pk1])
    partial = _edge_kernel(table, pk_p)
    return _combine_call(partial)


# speedup vs baseline: 1.3289x; 1.0188x over previous
import functools

import jax
import jax.numpy as jnp
from jax import lax
from jax.experimental import pallas as pl
from jax.experimental.pallas import tpu as pltpu
from jax.experimental.pallas import tpu_sc as plsc

N = 10000
F = 128
T = 4
E = 320000

NC = 2
NS = 16
NW = NC * NS
CH = 96
CPW0 = 140
CPW1 = 70
CPW_MAX = max(CPW0, CPW1)
CNT0 = CPW0 * CH
E0 = NS * CNT0
CNT1 = (E - E0) // NS
DUMMY = N << 16
A_ROWS = 10112
RPT = A_ROWS // NS
CC = 96
GA = 10


def _prep_body(h_ref, w_ref, b_ref, src_ref, et_ref, dst_ref, tab_ref, pk_ref):
    hb = h_ref[...]
    for t in range(T):
        tab_ref[t] = lax.dot_general(
            hb, w_ref[t], (((1,), (1,)), ((), ())),
            preferred_element_type=jnp.float32) + b_ref[t]
    pk_ref[...] = (et_ref[...] * N + src_ref[...]) | (dst_ref[...] << 16)


_prep_call = pl.pallas_call(
    _prep_body,
    grid=(GA,),
    in_specs=[
        pl.BlockSpec((N // GA, F), lambda i: (i, 0)),
        pl.BlockSpec((T, F, F), lambda i: (0, 0, 0)),
        pl.BlockSpec((T, F), lambda i: (0, 0)),
        pl.BlockSpec((1, 1, E // GA), lambda i: (i, 0, 0)),
        pl.BlockSpec((1, 1, E // GA), lambda i: (i, 0, 0)),
        pl.BlockSpec((1, 1, E // GA), lambda i: (i, 0, 0)),
    ],
    out_specs=[
        pl.BlockSpec((T, N // GA, F), lambda i: (0, i, 0)),
        pl.BlockSpec((1, 1, E // GA), lambda i: (i, 0, 0)),
    ],
    out_shape=[
        jax.ShapeDtypeStruct((T, N, F), jnp.float32),
        jax.ShapeDtypeStruct((GA, 1, E // GA), jnp.int32),
    ],
)


@functools.partial(
    pl.kernel,
    out_type=jax.ShapeDtypeStruct((NC, A_ROWS, F), jnp.float32),
    mesh=plsc.VectorSubcoreMesh(core_axis_name="c", subcore_axis_name="s"),
    scratch_types=[
        pltpu.VMEM((CPW_MAX * CH,), jnp.int32),
        pltpu.VMEM((2, CH), jnp.int32),
        pltpu.VMEM((2, CH), jnp.int32),
        pltpu.VMEM((CH, F), jnp.float32),
        pltpu.VMEM((CH, F), jnp.float32),
        pltpu.VMEM_SHARED((A_ROWS, F), jnp.float32),
        pltpu.SemaphoreType.DMA,
        pltpu.SemaphoreType.DMA,
        pltpu.SemaphoreType.DMA,
    ],
)
def _edge_kernel(tab_hbm, pk_hbm, out_hbm,
                 pk_v, idxb, dstb, rows0, rows1, acc_s, gsem0, gsem1, ssem):
    cid = lax.axis_index("c")
    sid = lax.axis_index("s")
    base = sid * RPT
    nfull = RPT // CC
    rem = RPT - nfull * CC

    @pl.when(cid == 0)
    def _():
        pltpu.async_copy(pk_hbm.at[pl.ds(sid * CNT0, CNT0)], pk_v, gsem0)

    @pl.when(cid == 1)
    def _():
        pltpu.async_copy(pk_hbm.at[pl.ds(E0 + sid * CNT1, CNT1)],
                         pk_v.at[pl.ds(0, CNT1)], gsem0)
        for t in range((CPW1 * CH - CNT1) // 16):
            pk_v[pl.ds(CNT1 + t * 16, 16)] = jnp.full((16,), DUMMY, jnp.int32)

    def _zrow(i, carry):
        for j in range(F // 16):
            rows0[i, pl.ds(j * 16, 16)] = jnp.zeros((16,), jnp.float32)
        return carry
    lax.fori_loop(0, CC, _zrow, 0)
    for m in range(nfull):
        pltpu.async_copy(rows0, acc_s.at[pl.ds(base + m * CC, CC)], ssem)
    pltpu.async_copy(rows0.at[pl.ds(0, rem)],
                     acc_s.at[pl.ds(base + nfull * CC, rem)], ssem)
    for m in range(nfull):
        pltpu.make_async_copy(
            rows0, acc_s.at[pl.ds(base + m * CC, CC)], ssem).wait()
    pltpu.make_async_copy(rows0.at[pl.ds(0, rem)],
                          acc_s.at[pl.ds(base + nfull * CC, rem)], ssem).wait()

    @pl.when(cid == 0)
    def _():
        pltpu.make_async_copy(pk_hbm.at[pl.ds(sid * CNT0, CNT0)],
                              pk_v, gsem0).wait()

    @pl.when(cid == 1)
    def _():
        pltpu.make_async_copy(pk_hbm.at[pl.ds(E0 + sid * CNT1, CNT1)],
                              pk_v.at[pl.ds(0, CNT1)], gsem0).wait()
    plsc.subcore_barrier()

    def _pair(k, carry):
        e0 = 2 * k * CH
        for q in range(2):
            for j in range(CH // 16):
                sl = pl.ds(j * 16, 16)
                pk = pk_v[pl.ds(e0 + q * CH + j * 16, 16)]
                idxb[q, sl] = pk & 0xFFFF
                dstb[q, sl] = lax.shift_right_logical(pk, 16)
        pltpu.async_copy(tab_hbm.at[idxb.at[0]], rows0, gsem0)
        pltpu.async_copy(tab_hbm.at[idxb.at[1]], rows1, gsem1)
        pltpu.make_async_copy(tab_hbm.at[idxb.at[0]], rows0, gsem0).wait()
        pltpu.async_copy(rows0, acc_s.at[dstb.at[0]], ssem, add=True)
        pltpu.make_async_copy(tab_hbm.at[idxb.at[1]], rows1, gsem1).wait()
        pltpu.sync_copy(rows1, acc_s.at[dstb.at[1]], add=True)
        pltpu.make_async_copy(rows0, acc_s.at[dstb.at[0]], ssem).wait()
        return carry
    npairs = lax.select(cid == 0, CPW0 // 2, CPW1 // 2)
    lax.fori_loop(0, npairs, _pair, 0)
    plsc.subcore_barrier()

    bufs = [rows0, rows1]
    szs = [CC] * nfull + [rem]
    for m, sz in enumerate(szs):
        buf = bufs[m % 2]
        r0 = base + m * CC
        if m >= 2:
            pltpu.make_async_copy(
                buf.at[pl.ds(0, szs[m - 2])],
                out_hbm.at[cid, pl.ds(base + (m - 2) * CC, szs[m - 2])],
                ssem).wait()
        pltpu.sync_copy(acc_s.at[pl.ds(r0, sz)], buf.at[pl.ds(0, sz)])
        pltpu.async_copy(buf.at[pl.ds(0, sz)],
                         out_hbm.at[cid, pl.ds(r0, sz)], ssem)
    for m in (len(szs) - 2, len(szs) - 1):
        pltpu.make_async_copy(
            bufs[m % 2].at[pl.ds(0, szs[m])],
            out_hbm.at[cid, pl.ds(base + m * CC, szs[m])], ssem).wait()


def _combine_body(p_ref, o_ref):
    o_ref[...] = p_ref[0] + p_ref[1]


_combine_call = pl.pallas_call(
    _combine_body,
    grid=(GA,),
    in_specs=[pl.BlockSpec((NC, N // GA, F), lambda i: (0, i, 0))],
    out_specs=pl.BlockSpec((N // GA, F), lambda i: (i, 0)),
    out_shape=jax.ShapeDtypeStruct((N, F), jnp.float32),
)


def kernel(feat, edge_index, etypes, W, b):
    src = edge_index[0]
    dst = edge_index[1]
    tab4, pk3 = _prep_call(
        feat, W, b,
        src.reshape(GA, 1, E // GA), etypes.reshape(GA, 1, E // GA),
        dst.reshape(GA, 1, E // GA))
    table = tab4.reshape(T * N, F)
    partial = _edge_kernel(table, pk3.reshape(-1))
    return _combine_call(partial)


# speedup vs baseline: 1.3694x; 1.0305x over previous
import functools

import jax
import jax.numpy as jnp
from jax import lax
from jax.experimental import pallas as pl
from jax.experimental.pallas import tpu as pltpu
from jax.experimental.pallas import tpu_sc as plsc

N = 10000
F = 128
T = 4
E = 320000

NC = 2
NS = 16
NW = NC * NS
CH = 96
CPW0 = 134
CPW1 = 76
CPW_MAX = max(CPW0, CPW1)
CNT0 = CPW0 * CH
E0 = NS * CNT0
CNT1 = (E - E0) // NS
DUMMY = N << 16
A_ROWS = 10112
RPT = A_ROWS // NS
CC = 96
GA = 10


def _prep_body(h_ref, w_ref, b_ref, src_ref, et_ref, dst_ref, tab_ref, pk_ref):
    hb = h_ref[...]
    for t in range(T):
        tab_ref[t] = lax.dot_general(
            hb, w_ref[t], (((1,), (1,)), ((), ())),
            preferred_element_type=jnp.float32) + b_ref[t]
    pk_ref[...] = (et_ref[...] * N + src_ref[...]) | (dst_ref[...] << 16)


_prep_call = pl.pallas_call(
    _prep_body,
    grid=(GA,),
    in_specs=[
        pl.BlockSpec((N // GA, F), lambda i: (i, 0)),
        pl.BlockSpec((T, F, F), lambda i: (0, 0, 0)),
        pl.BlockSpec((T, F), lambda i: (0, 0)),
        pl.BlockSpec((1, 1, E // GA), lambda i: (i, 0, 0)),
        pl.BlockSpec((1, 1, E // GA), lambda i: (i, 0, 0)),
        pl.BlockSpec((1, 1, E // GA), lambda i: (i, 0, 0)),
    ],
    out_specs=[
        pl.BlockSpec((T, N // GA, F), lambda i: (0, i, 0)),
        pl.BlockSpec((1, 1, E // GA), lambda i: (i, 0, 0)),
    ],
    out_shape=[
        jax.ShapeDtypeStruct((T, N, F), jnp.float32),
        jax.ShapeDtypeStruct((GA, 1, E // GA), jnp.int32),
    ],
)


@functools.partial(
    pl.kernel,
    out_type=jax.ShapeDtypeStruct((NC, A_ROWS, F), jnp.float32),
    mesh=plsc.VectorSubcoreMesh(core_axis_name="c", subcore_axis_name="s"),
    scratch_types=[
        pltpu.VMEM((CPW_MAX * CH,), jnp.int32),
        pltpu.VMEM((2, CH), jnp.int32),
        pltpu.VMEM((2, CH), jnp.int32),
        pltpu.VMEM((CH, F), jnp.float32),
        pltpu.VMEM((CH, F), jnp.float32),
        pltpu.VMEM_SHARED((A_ROWS, F), jnp.float32),
        pltpu.SemaphoreType.DMA,
        pltpu.SemaphoreType.DMA,
        pltpu.SemaphoreType.DMA,
    ],
)
def _edge_kernel(tab_hbm, pk_hbm, out_hbm,
                 pk_v, idxb, dstb, rows0, rows1, acc_s, gsem0, gsem1, ssem):
    cid = lax.axis_index("c")
    sid = lax.axis_index("s")
    base = sid * RPT
    nfull = RPT // CC
    rem = RPT - nfull * CC

    @pl.when(cid == 0)
    def _():
        pltpu.async_copy(pk_hbm.at[pl.ds(sid * CNT0, CNT0)], pk_v, gsem0)

    @pl.when(cid == 1)
    def _():
        pltpu.async_copy(pk_hbm.at[pl.ds(E0 + sid * CNT1, CNT1)],
                         pk_v.at[pl.ds(0, CNT1)], gsem0)
        for t in range((CPW1 * CH - CNT1) // 16):
            pk_v[pl.ds(CNT1 + t * 16, 16)] = jnp.full((16,), DUMMY, jnp.int32)

    def _zrow(i, carry):
        for j in range(F // 16):
            rows0[i, pl.ds(j * 16, 16)] = jnp.zeros((16,), jnp.float32)
        return carry
    lax.fori_loop(0, CC, _zrow, 0)
    for m in range(nfull):
        pltpu.async_copy(rows0, acc_s.at[pl.ds(base + m * CC, CC)], ssem)
    pltpu.async_copy(rows0.at[pl.ds(0, rem)],
                     acc_s.at[pl.ds(base + nfull * CC, rem)], ssem)
    for m in range(nfull):
        pltpu.make_async_copy(
            rows0, acc_s.at[pl.ds(base + m * CC, CC)], ssem).wait()
    pltpu.make_async_copy(rows0.at[pl.ds(0, rem)],
                          acc_s.at[pl.ds(base + nfull * CC, rem)], ssem).wait()

    @pl.when(cid == 0)
    def _():
        pltpu.make_async_copy(pk_hbm.at[pl.ds(sid * CNT0, CNT0)],
                              pk_v, gsem0).wait()

    @pl.when(cid == 1)
    def _():
        pltpu.make_async_copy(pk_hbm.at[pl.ds(E0 + sid * CNT1, CNT1)],
                              pk_v.at[pl.ds(0, CNT1)], gsem0).wait()
    plsc.subcore_barrier()

    def _pair(k, carry):
        e0 = 2 * k * CH
        for q in range(2):
            for j in range(CH // 16):
                sl = pl.ds(j * 16, 16)
                pk = pk_v[pl.ds(e0 + q * CH + j * 16, 16)]
                idxb[q, sl] = pk & 0xFFFF
                dstb[q, sl] = lax.shift_right_logical(pk, 16)
        pltpu.async_copy(tab_hbm.at[idxb.at[0]], rows0, gsem0)
        pltpu.async_copy(tab_hbm.at[idxb.at[1]], rows1, gsem1)
        pltpu.make_async_copy(tab_hbm.at[idxb.at[0]], rows0, gsem0).wait()
        pltpu.async_copy(rows0, acc_s.at[dstb.at[0]], ssem, add=True)
        pltpu.make_async_copy(tab_hbm.at[idxb.at[1]], rows1, gsem1).wait()
        pltpu.sync_copy(rows1, acc_s.at[dstb.at[1]], add=True)
        pltpu.make_async_copy(rows0, acc_s.at[dstb.at[0]], ssem).wait()
        return carry
    npairs = lax.select(cid == 0, CPW0 // 2, CPW1 // 2)
    lax.fori_loop(0, npairs, _pair, 0)
    plsc.subcore_barrier()

    bufs = [rows0, rows1]
    szs = [CC] * nfull + [rem]
    for m, sz in enumerate(szs):
        buf = bufs[m % 2]
        r0 = base + m * CC
        if m >= 2:
            pltpu.make_async_copy(
                buf.at[pl.ds(0, szs[m - 2])],
                out_hbm.at[cid, pl.ds(base + (m - 2) * CC, szs[m - 2])],
                ssem).wait()
        pltpu.sync_copy(acc_s.at[pl.ds(r0, sz)], buf.at[pl.ds(0, sz)])
        pltpu.async_copy(buf.at[pl.ds(0, sz)],
                         out_hbm.at[cid, pl.ds(r0, sz)], ssem)
    for m in (len(szs) - 2, len(szs) - 1):
        pltpu.make_async_copy(
            bufs[m % 2].at[pl.ds(0, szs[m])],
            out_hbm.at[cid, pl.ds(base + m * CC, szs[m])], ssem).wait()


def _combine_body(p_ref, o_ref):
    o_ref[...] = p_ref[0] + p_ref[1]


_combine_call = pl.pallas_call(
    _combine_body,
    grid=(GA,),
    in_specs=[pl.BlockSpec((NC, N // GA, F), lambda i: (0, i, 0))],
    out_specs=pl.BlockSpec((N // GA, F), lambda i: (i, 0)),
    out_shape=jax.ShapeDtypeStruct((N, F), jnp.float32),
)


def kernel(feat, edge_index, etypes, W, b):
    src = edge_index[0]
    dst = edge_index[1]
    tab4, pk3 = _prep_call(
        feat, W, b,
        src.reshape(GA, 1, E // GA), etypes.reshape(GA, 1, E // GA),
        dst.reshape(GA, 1, E // GA))
    table = tab4.reshape(T * N, F)
    partial = _edge_kernel(table, pk3.reshape(-1))
    return _combine_call(partial)


# speedup vs baseline: 1.3829x; 1.0099x over previous
import functools

import jax
import jax.numpy as jnp
from jax import lax
from jax.experimental import pallas as pl
from jax.experimental.pallas import tpu as pltpu
from jax.experimental.pallas import tpu_sc as plsc

N = 10000
F = 128
T = 4
E = 320000

NC = 2
NS = 16
NW = NC * NS
CH = 96
CPW0 = 132
CPW1 = 78
CPW_MAX = max(CPW0, CPW1)
CNT0 = CPW0 * CH
E0 = NS * CNT0
CNT1 = (E - E0) // NS
DUMMY = N << 16
A_ROWS = 10112
RPT = A_ROWS // NS
CC = 96
GA = 10


def _prep_body(h_ref, w_ref, b_ref, src_ref, et_ref, dst_ref, tab_ref, pk_ref):
    hb = h_ref[...]
    for t in range(T):
        tab_ref[t] = lax.dot_general(
            hb, w_ref[t], (((1,), (1,)), ((), ())),
            preferred_element_type=jnp.float32) + b_ref[t]
    pk_ref[...] = (et_ref[...] * N + src_ref[...]) | (dst_ref[...] << 16)


_prep_call = pl.pallas_call(
    _prep_body,
    grid=(GA,),
    in_specs=[
        pl.BlockSpec((N // GA, F), lambda i: (i, 0)),
        pl.BlockSpec((T, F, F), lambda i: (0, 0, 0)),
        pl.BlockSpec((T, F), lambda i: (0, 0)),
        pl.BlockSpec((1, 1, E // GA), lambda i: (i, 0, 0)),
        pl.BlockSpec((1, 1, E // GA), lambda i: (i, 0, 0)),
        pl.BlockSpec((1, 1, E // GA), lambda i: (i, 0, 0)),
    ],
    out_specs=[
        pl.BlockSpec((T, N // GA, F), lambda i: (0, i, 0)),
        pl.BlockSpec((1, 1, E // GA), lambda i: (i, 0, 0)),
    ],
    out_shape=[
        jax.ShapeDtypeStruct((T, N, F), jnp.float32),
        jax.ShapeDtypeStruct((GA, 1, E // GA), jnp.int32),
    ],
)


@functools.partial(
    pl.kernel,
    out_type=jax.ShapeDtypeStruct((NC, A_ROWS, F), jnp.float32),
    mesh=plsc.VectorSubcoreMesh(core_axis_name="c", subcore_axis_name="s"),
    scratch_types=[
        pltpu.VMEM((CPW_MAX * CH,), jnp.int32),
        pltpu.VMEM((2, CH), jnp.int32),
        pltpu.VMEM((2, CH), jnp.int32),
        pltpu.VMEM((CH, F), jnp.float32),
        pltpu.VMEM((CH, F), jnp.float32),
        pltpu.VMEM_SHARED((A_ROWS, F), jnp.float32),
        pltpu.SemaphoreType.DMA,
        pltpu.SemaphoreType.DMA,
        pltpu.SemaphoreType.DMA,
    ],
)
def _edge_kernel(tab_hbm, pk_hbm, out_hbm,
                 pk_v, idxb, dstb, rows0, rows1, acc_s, gsem0, gsem1, ssem):
    cid = lax.axis_index("c")
    sid = lax.axis_index("s")
    base = sid * RPT
    nfull = RPT // CC
    rem = RPT - nfull * CC

    @pl.when(cid == 0)
    def _():
        pltpu.async_copy(pk_hbm.at[pl.ds(sid * CNT0, CNT0)], pk_v, gsem0)

    @pl.when(cid == 1)
    def _():
        pltpu.async_copy(pk_hbm.at[pl.ds(E0 + sid * CNT1, CNT1)],
                         pk_v.at[pl.ds(0, CNT1)], gsem0)
        for t in range((CPW1 * CH - CNT1) // 16):
            pk_v[pl.ds(CNT1 + t * 16, 16)] = jnp.full((16,), DUMMY, jnp.int32)

    def _zrow(i, carry):
        for j in range(F // 16):
            rows0[i, pl.ds(j * 16, 16)] = jnp.zeros((16,), jnp.float32)
        return carry
    lax.fori_loop(0, CC, _zrow, 0)
    for m in range(nfull):
        pltpu.async_copy(rows0, acc_s.at[pl.ds(base + m * CC, CC)], ssem)
    pltpu.async_copy(rows0.at[pl.ds(0, rem)],
                     acc_s.at[pl.ds(base + nfull * CC, rem)], ssem)
    for m in range(nfull):
        pltpu.make_async_copy(
            rows0, acc_s.at[pl.ds(base + m * CC, CC)], ssem).wait()
    pltpu.make_async_copy(rows0.at[pl.ds(0, rem)],
                          acc_s.at[pl.ds(base + nfull * CC, rem)], ssem).wait()

    @pl.when(cid == 0)
    def _():
        pltpu.make_async_copy(pk_hbm.at[pl.ds(sid * CNT0, CNT0)],
                              pk_v, gsem0).wait()

    @pl.when(cid == 1)
    def _():
        pltpu.make_async_copy(pk_hbm.at[pl.ds(E0 + sid * CNT1, CNT1)],
                              pk_v.at[pl.ds(0, CNT1)], gsem0).wait()
    plsc.subcore_barrier()

    def _pair(k, carry):
        e0 = 2 * k * CH
        for q in range(2):
            for j in range(CH // 16):
                sl = pl.ds(j * 16, 16)
                pk = pk_v[pl.ds(e0 + q * CH + j * 16, 16)]
                idxb[q, sl] = pk & 0xFFFF
                dstb[q, sl] = lax.shift_right_logical(pk, 16)
        pltpu.async_copy(tab_hbm.at[idxb.at[0]], rows0, gsem0)
        pltpu.async_copy(tab_hbm.at[idxb.at[1]], rows1, gsem1)
        pltpu.make_async_copy(tab_hbm.at[idxb.at[0]], rows0, gsem0).wait()
        pltpu.async_copy(rows0, acc_s.at[dstb.at[0]], ssem, add=True)
        pltpu.make_async_copy(tab_hbm.at[idxb.at[1]], rows1, gsem1).wait()
        pltpu.sync_copy(rows1, acc_s.at[dstb.at[1]], add=True)
        pltpu.make_async_copy(rows0, acc_s.at[dstb.at[0]], ssem).wait()
        return carry
    npairs = lax.select(cid == 0, CPW0 // 2, CPW1 // 2)
    lax.fori_loop(0, npairs, _pair, 0)
    plsc.subcore_barrier()

    bufs = [rows0, rows1]
    szs = [CC] * nfull + [rem]
    for m, sz in enumerate(szs):
        buf = bufs[m % 2]
        r0 = base + m * CC
        if m >= 2:
            pltpu.make_async_copy(
                buf.at[pl.ds(0, szs[m - 2])],
                out_hbm.at[cid, pl.ds(base + (m - 2) * CC, szs[m - 2])],
                ssem).wait()
        pltpu.sync_copy(acc_s.at[pl.ds(r0, sz)], buf.at[pl.ds(0, sz)])
        pltpu.async_copy(buf.at[pl.ds(0, sz)],
                         out_hbm.at[cid, pl.ds(r0, sz)], ssem)
    for m in (len(szs) - 2, len(szs) - 1):
        pltpu.make_async_copy(
            bufs[m % 2].at[pl.ds(0, szs[m])],
            out_hbm.at[cid, pl.ds(base + m * CC, szs[m])], ssem).wait()


def _combine_body(p_ref, o_ref):
    o_ref[...] = p_ref[0] + p_ref[1]


_combine_call = pl.pallas_call(
    _combine_body,
    grid=(GA,),
    in_specs=[pl.BlockSpec((NC, N // GA, F), lambda i: (0, i, 0))],
    out_specs=pl.BlockSpec((N // GA, F), lambda i: (i, 0)),
    out_shape=jax.ShapeDtypeStruct((N, F), jnp.float32),
)


def kernel(feat, edge_index, etypes, W, b):
    src = edge_index[0]
    dst = edge_index[1]
    tab4, pk3 = _prep_call(
        feat, W, b,
        src.reshape(GA, 1, E // GA), etypes.reshape(GA, 1, E // GA),
        dst.reshape(GA, 1, E // GA))
    table = tab4.reshape(T * N, F)
    partial = _edge_kernel(table, pk3.reshape(-1))
    return _combine_call(partial)


# speedup vs baseline: 1.4103x; 1.0198x over previous
import functools

import jax
import jax.numpy as jnp
from jax import lax
from jax.experimental import pallas as pl
from jax.experimental.pallas import tpu as pltpu
from jax.experimental.pallas import tpu_sc as plsc

N = 10000
F = 128
T = 4
E = 320000

NC = 2
NS = 16
NW = NC * NS
CH = 96
CPW0 = 128
CPW1 = 82
CPW_MAX = max(CPW0, CPW1)
CNT0 = CPW0 * CH
E0 = NS * CNT0
CNT1 = (E - E0) // NS
DUMMY = N << 16
A_ROWS = 10112
RPT = A_ROWS // NS
CC = 96
GA = 10


def _prep_body(h_ref, w_ref, b_ref, src_ref, et_ref, dst_ref, tab_ref, pk_ref):
    hb = h_ref[...]
    for t in range(T):
        tab_ref[t] = lax.dot_general(
            hb, w_ref[t], (((1,), (1,)), ((), ())),
            preferred_element_type=jnp.float32) + b_ref[t]
    pk_ref[...] = (et_ref[...] * N + src_ref[...]) | (dst_ref[...] << 16)


_prep_call = pl.pallas_call(
    _prep_body,
    grid=(GA,),
    in_specs=[
        pl.BlockSpec((N // GA, F), lambda i: (i, 0)),
        pl.BlockSpec((T, F, F), lambda i: (0, 0, 0)),
        pl.BlockSpec((T, F), lambda i: (0, 0)),
        pl.BlockSpec((1, 1, E // GA), lambda i: (i, 0, 0)),
        pl.BlockSpec((1, 1, E // GA), lambda i: (i, 0, 0)),
        pl.BlockSpec((1, 1, E // GA), lambda i: (i, 0, 0)),
    ],
    out_specs=[
        pl.BlockSpec((T, N // GA, F), lambda i: (0, i, 0)),
        pl.BlockSpec((1, 1, E // GA), lambda i: (i, 0, 0)),
    ],
    out_shape=[
        jax.ShapeDtypeStruct((T, N, F), jnp.float32),
        jax.ShapeDtypeStruct((GA, 1, E // GA), jnp.int32),
    ],
)


@functools.partial(
    pl.kernel,
    out_type=jax.ShapeDtypeStruct((NC, A_ROWS, F), jnp.float32),
    mesh=plsc.VectorSubcoreMesh(core_axis_name="c", subcore_axis_name="s"),
    scratch_types=[
        pltpu.VMEM((CPW_MAX * CH,), jnp.int32),
        pltpu.VMEM((2, CH), jnp.int32),
        pltpu.VMEM((2, CH), jnp.int32),
        pltpu.VMEM((CH, F), jnp.float32),
        pltpu.VMEM((CH, F), jnp.float32),
        pltpu.VMEM_SHARED((A_ROWS, F), jnp.float32),
        pltpu.SemaphoreType.DMA,
        pltpu.SemaphoreType.DMA,
        pltpu.SemaphoreType.DMA,
    ],
)
def _edge_kernel(tab_hbm, pk_hbm, out_hbm,
                 pk_v, idxb, dstb, rows0, rows1, acc_s, gsem0, gsem1, ssem):
    cid = lax.axis_index("c")
    sid = lax.axis_index("s")
    base = sid * RPT
    nfull = RPT // CC
    rem = RPT - nfull * CC

    @pl.when(cid == 0)
    def _():
        pltpu.async_copy(pk_hbm.at[pl.ds(sid * CNT0, CNT0)], pk_v, gsem0)

    @pl.when(cid == 1)
    def _():
        pltpu.async_copy(pk_hbm.at[pl.ds(E0 + sid * CNT1, CNT1)],
                         pk_v.at[pl.ds(0, CNT1)], gsem0)
        for t in range((CPW1 * CH - CNT1) // 16):
            pk_v[pl.ds(CNT1 + t * 16, 16)] = jnp.full((16,), DUMMY, jnp.int32)

    def _zrow(i, carry):
        for j in range(F // 16):
            rows0[i, pl.ds(j * 16, 16)] = jnp.zeros((16,), jnp.float32)
        return carry
    lax.fori_loop(0, CC, _zrow, 0)
    for m in range(nfull):
        pltpu.async_copy(rows0, acc_s.at[pl.ds(base + m * CC, CC)], ssem)
    pltpu.async_copy(rows0.at[pl.ds(0, rem)],
                     acc_s.at[pl.ds(base + nfull * CC, rem)], ssem)
    for m in range(nfull):
        pltpu.make_async_copy(
            rows0, acc_s.at[pl.ds(base + m * CC, CC)], ssem).wait()
    pltpu.make_async_copy(rows0.at[pl.ds(0, rem)],
                          acc_s.at[pl.ds(base + nfull * CC, rem)], ssem).wait()

    @pl.when(cid == 0)
    def _():
        pltpu.make_async_copy(pk_hbm.at[pl.ds(sid * CNT0, CNT0)],
                              pk_v, gsem0).wait()

    @pl.when(cid == 1)
    def _():
        pltpu.make_async_copy(pk_hbm.at[pl.ds(E0 + sid * CNT1, CNT1)],
                              pk_v.at[pl.ds(0, CNT1)], gsem0).wait()
    plsc.subcore_barrier()

    def _pair(k, carry):
        e0 = 2 * k * CH
        for q in range(2):
            for j in range(CH // 16):
                sl = pl.ds(j * 16, 16)
                pk = pk_v[pl.ds(e0 + q * CH + j * 16, 16)]
                idxb[q, sl] = pk & 0xFFFF
                dstb[q, sl] = lax.shift_right_logical(pk, 16)
        pltpu.async_copy(tab_hbm.at[idxb.at[0]], rows0, gsem0)
        pltpu.async_copy(tab_hbm.at[idxb.at[1]], rows1, gsem1)
        pltpu.make_async_copy(tab_hbm.at[idxb.at[0]], rows0, gsem0).wait()
        pltpu.async_copy(rows0, acc_s.at[dstb.at[0]], ssem, add=True)
        pltpu.make_async_copy(tab_hbm.at[idxb.at[1]], rows1, gsem1).wait()
        pltpu.sync_copy(rows1, acc_s.at[dstb.at[1]], add=True)
        pltpu.make_async_copy(rows0, acc_s.at[dstb.at[0]], ssem).wait()
        return carry
    npairs = lax.select(cid == 0, CPW0 // 2, CPW1 // 2)
    lax.fori_loop(0, npairs, _pair, 0)
    plsc.subcore_barrier()

    bufs = [rows0, rows1]
    szs = [CC] * nfull + [rem]
    for m, sz in enumerate(szs):
        buf = bufs[m % 2]
        r0 = base + m * CC
        if m >= 2:
            pltpu.make_async_copy(
                buf.at[pl.ds(0, szs[m - 2])],
                out_hbm.at[cid, pl.ds(base + (m - 2) * CC, szs[m - 2])],
                ssem).wait()
        pltpu.sync_copy(acc_s.at[pl.ds(r0, sz)], buf.at[pl.ds(0, sz)])
        pltpu.async_copy(buf.at[pl.ds(0, sz)],
                         out_hbm.at[cid, pl.ds(r0, sz)], ssem)
    for m in (len(szs) - 2, len(szs) - 1):
        pltpu.make_async_copy(
            bufs[m % 2].at[pl.ds(0, szs[m])],
            out_hbm.at[cid, pl.ds(base + m * CC, szs[m])], ssem).wait()


def _combine_body(p_ref, o_ref):
    o_ref[...] = p_ref[0] + p_ref[1]


_combine_call = pl.pallas_call(
    _combine_body,
    grid=(GA,),
    in_specs=[pl.BlockSpec((NC, N // GA, F), lambda i: (0, i, 0))],
    out_specs=pl.BlockSpec((N // GA, F), lambda i: (i, 0)),
    out_shape=jax.ShapeDtypeStruct((N, F), jnp.float32),
)


def kernel(feat, edge_index, etypes, W, b):
    src = edge_index[0]
    dst = edge_index[1]
    tab4, pk3 = _prep_call(
        feat, W, b,
        src.reshape(GA, 1, E // GA), etypes.reshape(GA, 1, E // GA),
        dst.reshape(GA, 1, E // GA))
    table = tab4.reshape(T * N, F)
    partial = _edge_kernel(table, pk3.reshape(-1))
    return _combine_call(partial)
